# Initial kernel scaffold; baseline (speedup 1.0000x reference)
import jax
import jax.numpy as jnp
from jax import lax
from jax.experimental import pallas as pl
from jax.experimental.pallas import tpu as pltpu
from jax.experimental.pallas import tpu_sc as plsc

_VOCAB = 100000
_LEN = 50
_DIM = 64
_BATCH = 4096
_DP = 16
_NC, _NS = 2, 16
_NW = _NC * _NS
_BPW = _BATCH // _NW


def _proj_body(t_ref, w_ref, o_ref):
    o_ref[...] = jnp.dot(t_ref[...], w_ref[...],
                         preferred_element_type=jnp.float32)


def _project(table, w_pad):
    blk = 2000
    return pl.pallas_call(
        _proj_body,
        grid=(_VOCAB // blk,),
        in_specs=[
            pl.BlockSpec((blk, _DIM), lambda i: (i, 0)),
            pl.BlockSpec((_DIM, _DP), lambda i: (0, 0)),
        ],
        out_specs=pl.BlockSpec((blk, _DP), lambda i: (i, 0)),
        out_shape=jax.ShapeDtypeStruct((_VOCAB, _DP), jnp.float32),
    )(table, w_pad)


def _sc_body(idx_hbm, p_hbm, bias_hbm, out_hbm, idx_v, rows_v, out_v, bias_v):
    wid = lax.axis_index("s") * _NC + lax.axis_index("c")
    pltpu.sync_copy(idx_hbm.at[wid], idx_v)
    pltpu.sync_copy(bias_hbm, bias_v)
    bias = bias_v[...]

    def step(m, carry):
        pltpu.sync_copy(p_hbm.at[idx_v.at[m]], rows_v)

        def acc_step(t, acc):
            return acc + rows_v[t]

        acc = lax.fori_loop(0, _LEN, acc_step,
                            jnp.zeros((_DP,), jnp.float32))
        y = acc * (1.0 / _LEN) + bias
        sq = jnp.maximum(jnp.sum(y * y), 1e-12)
        sqv = jnp.broadcast_to(sq, (_DP,))
        bits = lax.bitcast_convert_type(sqv, jnp.int32)
        bits = 0x5F3759DF - lax.shift_right_logical(bits, 1)
        r = lax.bitcast_convert_type(bits, jnp.float32)
        for _ in range(3):
            r = r * (1.5 - 0.5 * sqv * r * r)
        out_v[m] = y * r
        return carry

    lax.fori_loop(0, _BPW, step, 0)
    pltpu.sync_copy(out_v, out_hbm.at[pl.ds(wid * _BPW, _BPW)])


def kernel(inputs, table, W, b):
    w_pad = jnp.zeros((_DIM, _DP), jnp.float32).at[:, :3].set(W)
    b_pad = jnp.zeros((_DP,), jnp.float32).at[:3].set(b)
    p = _project(table, w_pad)
    idx = inputs.astype(jnp.int32).reshape(_NW, _BPW, _LEN)
    mesh = plsc.VectorSubcoreMesh(core_axis_name="c", subcore_axis_name="s")
    out = pl.kernel(
        _sc_body,
        out_type=jax.ShapeDtypeStruct((_BATCH, _DP), jnp.float32),
        mesh=mesh,
        scratch_types=[
            pltpu.VMEM((_BPW, _LEN), jnp.int32),
            pltpu.VMEM((_LEN, _DP), jnp.float32),
            pltpu.VMEM((_BPW, _DP), jnp.float32),
            pltpu.VMEM((_DP,), jnp.float32),
        ],
    )(idx, p, b_pad)
    return out[:, :3]


# speedup vs baseline: 3.9174x; 3.9174x over previous
import jax
import jax.numpy as jnp
from jax import lax
from jax.experimental import pallas as pl
from jax.experimental.pallas import tpu as pltpu
from jax.experimental.pallas import tpu_sc as plsc

_VOCAB = 100000
_LEN = 50
_DIM = 64
_BATCH = 4096
_DP = 16
_NC, _NS = 2, 16
_NW = _NC * _NS
_BPW = _BATCH // _NW


def _proj_body(t_ref, w_ref, o_ref):
    o_ref[...] = jnp.dot(t_ref[...], w_ref[...],
                         preferred_element_type=jnp.float32,
                         precision=lax.Precision.HIGHEST)


def _project(table, w_pad):
    blk = 2000
    return pl.pallas_call(
        _proj_body,
        grid=(_VOCAB // blk,),
        in_specs=[
            pl.BlockSpec((blk, _DIM), lambda i: (i, 0)),
            pl.BlockSpec((_DIM, _DP), lambda i: (0, 0)),
        ],
        out_specs=pl.BlockSpec((blk, _DP), lambda i: (i, 0)),
        out_shape=jax.ShapeDtypeStruct((_VOCAB, _DP), jnp.float32),
    )(table, w_pad)


def _sc_body(idx_hbm, p_hbm, bias_hbm, out_hbm, idx_v, rows_v, out_v, bias_v):
    wid = lax.axis_index("s") * _NC + lax.axis_index("c")
    pltpu.sync_copy(idx_hbm.at[wid], idx_v)
    pltpu.sync_copy(bias_hbm, bias_v)
    bias = bias_v[...]

    def step(m, carry):
        pltpu.sync_copy(p_hbm.at[idx_v.at[m]], rows_v)

        def acc_step(t, acc):
            return acc + rows_v[t]

        acc = lax.fori_loop(0, _LEN, acc_step,
                            jnp.zeros((_DP,), jnp.float32))
        y = acc * (1.0 / _LEN) + bias
        sq = jnp.maximum(y[0] * y[0] + y[1] * y[1] + y[2] * y[2],
                         jnp.float32(1e-12))
        bits = lax.bitcast_convert_type(sq, jnp.int32)
        bits = 0x5F3759DF - lax.shift_right_logical(bits, 1)
        r = lax.bitcast_convert_type(bits, jnp.float32)
        for _ in range(3):
            r = r * (1.5 - 0.5 * sq * r * r)
        out_v[m] = y * r
        return carry

    lax.fori_loop(0, _BPW, step, 0)
    pltpu.sync_copy(out_v, out_hbm.at[pl.ds(wid * _BPW, _BPW)])


def kernel(inputs, table, W, b):
    w_pad = jnp.zeros((_DIM, _DP), jnp.float32).at[:, :3].set(W)
    b_pad = jnp.zeros((_DP,), jnp.float32).at[:3].set(b)
    p = _project(table, w_pad)
    idx = inputs.astype(jnp.int32).reshape(_NW, _BPW, _LEN)
    mesh = plsc.VectorSubcoreMesh(core_axis_name="c", subcore_axis_name="s")
    out = pl.kernel(
        _sc_body,
        out_type=jax.ShapeDtypeStruct((_BATCH, _DP), jnp.float32),
        mesh=mesh,
        compiler_params=pltpu.CompilerParams(use_tc_tiling_on_sc=False),
        scratch_types=[
            pltpu.VMEM((_BPW, _LEN), jnp.int32),
            pltpu.VMEM((_LEN, _DP), jnp.float32),
            pltpu.VMEM((_BPW, _DP), jnp.float32),
            pltpu.VMEM((_DP,), jnp.float32),
        ],
    )(idx, p, b_pad)
    return out[:, :3]


# speedup vs baseline: 5.9044x; 1.5072x over previous
import jax
import jax.numpy as jnp
from jax import lax
from jax.experimental import pallas as pl
from jax.experimental.pallas import tpu as pltpu
from jax.experimental.pallas import tpu_sc as plsc

_VOCAB = 100000
_LEN = 50
_DIM = 64
_BATCH = 4096
_DP = 16
_NC, _NS = 2, 16
_NW = _NC * _NS
_BPW = _BATCH // _NW


def _proj_body(t_ref, w_ref, o_ref):
    o_ref[...] = jnp.dot(t_ref[...], w_ref[...],
                         preferred_element_type=jnp.float32,
                         precision=lax.Precision.HIGHEST)


def _project(table, w_pad):
    blk = 2000
    return pl.pallas_call(
        _proj_body,
        grid=(_VOCAB // blk,),
        in_specs=[
            pl.BlockSpec((blk, _DIM), lambda i: (i, 0)),
            pl.BlockSpec((_DIM, _DP), lambda i: (0, 0)),
        ],
        out_specs=pl.BlockSpec((blk, _DP), lambda i: (i, 0)),
        out_shape=jax.ShapeDtypeStruct((_VOCAB, _DP), jnp.float32),
    )(table, w_pad)


_NCHUNK = 50
_CPW = 128
_WIN = 8


def _sc_body(idx_hbm, p_hbm, bias_hbm, out_hbm, idx_v, rows_v, out_v, bias_v,
             sem):
    wid = lax.axis_index("s") * _NC + lax.axis_index("c")
    pltpu.sync_copy(idx_hbm.at[wid], idx_v)
    pltpu.sync_copy(bias_hbm, bias_v)
    bias = bias_v[...]

    def fire(j, c):
        pltpu.async_copy(p_hbm.at[idx_v.at[j]], rows_v.at[j], sem)
        return c

    def fire_drain(j, c):
        pltpu.async_copy(p_hbm.at[idx_v.at[j + _WIN]], rows_v.at[j + _WIN],
                         sem)
        pltpu.make_async_copy(p_hbm.at[idx_v.at[j]], rows_v.at[j], sem).wait()
        return c

    def drain(j, c):
        pltpu.make_async_copy(p_hbm.at[idx_v.at[j]], rows_v.at[j], sem).wait()
        return c

    lax.fori_loop(0, _WIN, fire, 0)
    lax.fori_loop(0, _NCHUNK - _WIN, fire_drain, 0)
    lax.fori_loop(_NCHUNK - _WIN, _NCHUNK, drain, 0)

    def step(m, carry):
        base = m * _LEN
        part = [None] * 4
        for t in range(_LEN):
            r = base + t
            v = rows_v[lax.shift_right_logical(r, 7), r & 127]
            k = t % 4
            part[k] = v if part[k] is None else part[k] + v
        acc = (part[0] + part[1]) + (part[2] + part[3])
        y = acc * (1.0 / _LEN) + bias
        sq = jnp.maximum(y[0] * y[0] + y[1] * y[1] + y[2] * y[2],
                         jnp.float32(1e-12))
        bits = lax.bitcast_convert_type(sq, jnp.int32)
        bits = 0x5F3759DF - lax.shift_right_logical(bits, 1)
        r = lax.bitcast_convert_type(bits, jnp.float32)
        for _ in range(3):
            r = r * (1.5 - 0.5 * sq * r * r)
        out_v[m] = y * r
        return carry

    lax.fori_loop(0, _BPW, step, 0)
    pltpu.sync_copy(out_v, out_hbm.at[pl.ds(wid * _BPW, _BPW)])


def kernel(inputs, table, W, b):
    w_pad = jnp.zeros((_DIM, _DP), jnp.float32).at[:, :3].set(W)
    b_pad = jnp.zeros((_DP,), jnp.float32).at[:3].set(b)
    p = _project(table, w_pad)
    idx = inputs.astype(jnp.int32).reshape(_NW, _NCHUNK, _CPW)
    mesh = plsc.VectorSubcoreMesh(core_axis_name="c", subcore_axis_name="s")
    out = pl.kernel(
        _sc_body,
        out_type=jax.ShapeDtypeStruct((_BATCH, _DP), jnp.float32),
        mesh=mesh,
        compiler_params=pltpu.CompilerParams(use_tc_tiling_on_sc=False),
        scratch_types=[
            pltpu.VMEM((_NCHUNK, _CPW), jnp.int32),
            pltpu.VMEM((_NCHUNK, _CPW, _DP), jnp.float32),
            pltpu.VMEM((_BPW, _DP), jnp.float32),
            pltpu.VMEM((_DP,), jnp.float32),
            pltpu.SemaphoreType.DMA,
        ],
    )(idx, p, b_pad)
    return out[:, :3]


# speedup vs baseline: 9.9444x; 1.6842x over previous
import jax
import jax.numpy as jnp
from jax import lax
from jax.experimental import pallas as pl
from jax.experimental.pallas import tpu as pltpu
from jax.experimental.pallas import tpu_sc as plsc

_VOCAB = 100000
_LEN = 50
_DIM = 64
_BATCH = 4096
_DP = 16
_NC, _NS = 2, 16
_NW = _NC * _NS
_BPW = _BATCH // _NW


_BLK = 4000
_NBLK = _VOCAB // _BLK
_G = _BLK // 8


def _proj_body(t_ref, w_ref, o_ref):
    acc = jnp.zeros((_G, 8 * _DP), jnp.float32)
    for r in range(8):
        acc = acc + jnp.dot(t_ref[0, :, r, :], w_ref[r],
                            preferred_element_type=jnp.float32)
    o_ref[...] = acc.reshape(1, _G, 8 * _DP)


def _project(table, w_blk):
    return pl.pallas_call(
        _proj_body,
        grid=(_NBLK,),
        in_specs=[
            pl.BlockSpec((1, _G, 8, _DIM), lambda i: (i, 0, 0, 0)),
            pl.BlockSpec((8, _DIM, 8 * _DP), lambda i: (0, 0, 0)),
        ],
        out_specs=pl.BlockSpec((1, _G, 8 * _DP), lambda i: (i, 0, 0)),
        out_shape=jax.ShapeDtypeStruct((_NBLK, _G, 8 * _DP), jnp.float32),
    )(table.reshape(_NBLK, _G, 8, _DIM), w_blk)


_NCHUNK = 50
_CPW = 128
_WIN = 8


def _sc_body(idx_hbm, p_hbm, bias_hbm, out_hbm, idx_v, rows_v, out_v, bias_v,
             sem):
    wid = lax.axis_index("s") * _NC + lax.axis_index("c")
    pltpu.sync_copy(idx_hbm.at[wid], idx_v)
    pltpu.sync_copy(bias_hbm, bias_v)
    bias = bias_v[...]

    def fire(j, c):
        pltpu.async_copy(p_hbm.at[idx_v.at[j]], rows_v.at[j], sem)
        return c

    def fire_drain(j, c):
        pltpu.async_copy(p_hbm.at[idx_v.at[j + _WIN]], rows_v.at[j + _WIN],
                         sem)
        pltpu.make_async_copy(p_hbm.at[idx_v.at[j]], rows_v.at[j], sem).wait()
        return c

    def drain(j, c):
        pltpu.make_async_copy(p_hbm.at[idx_v.at[j]], rows_v.at[j], sem).wait()
        return c

    lax.fori_loop(0, _WIN, fire, 0)
    lax.fori_loop(0, _NCHUNK - _WIN, fire_drain, 0)
    lax.fori_loop(_NCHUNK - _WIN, _NCHUNK, drain, 0)

    def step(m, carry):
        base = m * _LEN
        part = [None] * 4
        for t in range(_LEN):
            r = base + t
            v = rows_v[lax.shift_right_logical(r, 7), r & 127]
            k = t % 4
            part[k] = v if part[k] is None else part[k] + v
        acc = (part[0] + part[1]) + (part[2] + part[3])
        y = acc * (1.0 / _LEN) + bias
        sq = jnp.maximum(y[0] * y[0] + y[1] * y[1] + y[2] * y[2],
                         jnp.float32(1e-12))
        bits = lax.bitcast_convert_type(sq, jnp.int32)
        bits = 0x5F3759DF - lax.shift_right_logical(bits, 1)
        r = lax.bitcast_convert_type(bits, jnp.float32)
        for _ in range(3):
            r = r * (1.5 - 0.5 * sq * r * r)
        out_v[m] = y * r
        return carry

    lax.fori_loop(0, _BPW, step, 0)
    pltpu.sync_copy(out_v, out_hbm.at[pl.ds(wid * _BPW, _BPW)])


def kernel(inputs, table, W, b):
    import os as _os
    _diag = _os.environ.get("KDIAG", "")
    w_blk = jnp.zeros((8, _DIM, 8 * _DP), jnp.float32)
    for _r in range(8):
        w_blk = w_blk.at[_r, :, 16 * _r:16 * _r + 3].set(W)
    if _diag == "tcfull":
        return _project(table, w_blk)
    if _diag == "reshape":
        return table.reshape(_VOCAB // 8, 8 * _DIM)
    if _diag == "copy":
        def _cp(t_ref, o_ref):
            o_ref[...] = t_ref[...]
        blk = 10000
        return pl.pallas_call(
            _cp,
            grid=(_VOCAB // blk,),
            in_specs=[pl.BlockSpec((blk, _DIM), lambda i: (i, 0))],
            out_specs=pl.BlockSpec((blk, _DIM), lambda i: (i, 0)),
            out_shape=jax.ShapeDtypeStruct((_VOCAB, _DIM), jnp.float32),
        )(table)
    if _diag == "sc":
        b_pad = jnp.zeros((_DP,), jnp.float32).at[:3].set(b)
        p = jnp.zeros((_VOCAB, _DP), jnp.float32)
        idx = inputs.astype(jnp.int32).reshape(_NW, _NCHUNK, _CPW)
        mesh = plsc.VectorSubcoreMesh(core_axis_name="c",
                                      subcore_axis_name="s")
        out = pl.kernel(
            _sc_body,
            out_type=jax.ShapeDtypeStruct((_BATCH, _DP), jnp.float32),
            mesh=mesh,
            compiler_params=pltpu.CompilerParams(use_tc_tiling_on_sc=False),
            scratch_types=[
                pltpu.VMEM((_NCHUNK, _CPW), jnp.int32),
                pltpu.VMEM((_NCHUNK, _CPW, _DP), jnp.float32),
                pltpu.VMEM((_BPW, _DP), jnp.float32),
                pltpu.VMEM((_DP,), jnp.float32),
                pltpu.SemaphoreType.DMA,
            ],
        )(idx, p, b_pad)
        return out[:, :3]
    b_pad = jnp.zeros((_DP,), jnp.float32).at[:3].set(b)
    p = _project(table, w_blk).reshape(_VOCAB, _DP)
    idx = inputs.astype(jnp.int32).reshape(_NW, _NCHUNK, _CPW)
    mesh = plsc.VectorSubcoreMesh(core_axis_name="c", subcore_axis_name="s")
    out = pl.kernel(
        _sc_body,
        out_type=jax.ShapeDtypeStruct((_BATCH, _DP), jnp.float32),
        mesh=mesh,
        compiler_params=pltpu.CompilerParams(use_tc_tiling_on_sc=False),
        scratch_types=[
            pltpu.VMEM((_NCHUNK, _CPW), jnp.int32),
            pltpu.VMEM((_NCHUNK, _CPW, _DP), jnp.float32),
            pltpu.VMEM((_BPW, _DP), jnp.float32),
            pltpu.VMEM((_DP,), jnp.float32),
            pltpu.SemaphoreType.DMA,
        ],
    )(idx, p, b_pad)
    return out[:, :3]


# speedup vs baseline: 10.4681x; 1.0527x over previous
import jax
import jax.numpy as jnp
from jax import lax
from jax.experimental import pallas as pl
from jax.experimental.pallas import tpu as pltpu
from jax.experimental.pallas import tpu_sc as plsc

_VOCAB = 100000
_LEN = 50
_DIM = 64
_BATCH = 4096
_DP = 16
_NC, _NS = 2, 16
_NW = _NC * _NS
_BPW = _BATCH // _NW


_BLK = 4000
_NBLK = _VOCAB // _BLK
_G = _BLK // 8


def _proj_body(t_ref, w_ref, o_ref):
    acc = jnp.zeros((_G, 8 * _DP), jnp.float32)
    for r in range(8):
        acc = acc + jnp.dot(t_ref[0, :, r, :], w_ref[r],
                            preferred_element_type=jnp.float32)
    o_ref[...] = acc.reshape(1, _G, 8 * _DP)


def _project(table, w_blk):
    return pl.pallas_call(
        _proj_body,
        grid=(_NBLK,),
        in_specs=[
            pl.BlockSpec((1, _G, 8, _DIM), lambda i: (i, 0, 0, 0)),
            pl.BlockSpec((8, _DIM, 8 * _DP), lambda i: (0, 0, 0)),
        ],
        out_specs=pl.BlockSpec((1, _G, 8 * _DP), lambda i: (i, 0, 0)),
        out_shape=jax.ShapeDtypeStruct((_NBLK, _G, 8 * _DP), jnp.float32),
    )(table.reshape(_NBLK, _G, 8, _DIM), w_blk)


_EPC = 2
_CPW = _EPC * _LEN
_NCHUNK = _BPW // _EPC
_WIN = 8


def _sc_body(idx_hbm, p_hbm, bias_hbm, out_hbm, idx_v, rows_v, out_v, bias_v,
             sem):
    wid = lax.axis_index("s") * _NC + lax.axis_index("c")
    pltpu.sync_copy(idx_hbm.at[wid], idx_v)
    pltpu.sync_copy(bias_hbm, bias_v)
    bias = bias_v[...]

    def _fire(j):
        pltpu.async_copy(p_hbm.at[idx_v.at[j]], rows_v.at[j], sem)

    def _drain(j):
        pltpu.make_async_copy(p_hbm.at[idx_v.at[j]], rows_v.at[j], sem).wait()

    def _finalize(y):
        sq = jnp.maximum(y[0] * y[0] + y[1] * y[1] + y[2] * y[2],
                         jnp.float32(1e-12))
        bits = lax.bitcast_convert_type(sq, jnp.int32)
        bits = 0x5F3759DF - lax.shift_right_logical(bits, 1)
        r = lax.bitcast_convert_type(bits, jnp.float32)
        for _ in range(3):
            r = r * (1.5 - 0.5 * sq * r * r)
        return y * r

    def _process(j):
        for e in range(_EPC):
            part = [None] * 4
            for t in range(_LEN):
                v = rows_v[j, e * _LEN + t]
                k = t % 4
                part[k] = v if part[k] is None else part[k] + v
            acc = (part[0] + part[1]) + (part[2] + part[3])
            y = acc * (1.0 / _LEN) + bias
            out_v[_EPC * j + e] = _finalize(y)

    def fire(j, c):
        _fire(j)
        return c

    def steady(j, c):
        _fire(j + _WIN)
        _drain(j)
        _process(j)
        return c

    def tail(j, c):
        _drain(j)
        _process(j)
        return c

    lax.fori_loop(0, _WIN, fire, 0)
    lax.fori_loop(0, _NCHUNK - _WIN, steady, 0)
    lax.fori_loop(_NCHUNK - _WIN, _NCHUNK, tail, 0)
    pltpu.sync_copy(out_v, out_hbm.at[pl.ds(wid * _BPW, _BPW)])


def kernel(inputs, table, W, b):
    import os as _os
    _diag = _os.environ.get("KDIAG", "")
    w_blk = jnp.zeros((8, _DIM, 8 * _DP), jnp.float32)
    for _r in range(8):
        w_blk = w_blk.at[_r, :, 16 * _r:16 * _r + 3].set(W)
    if _diag == "tcfull":
        return _project(table, w_blk)
    if _diag == "reshape":
        return table.reshape(_VOCAB // 8, 8 * _DIM)
    if _diag == "copy":
        def _cp(t_ref, o_ref):
            o_ref[...] = t_ref[...]
        blk = 10000
        return pl.pallas_call(
            _cp,
            grid=(_VOCAB // blk,),
            in_specs=[pl.BlockSpec((blk, _DIM), lambda i: (i, 0))],
            out_specs=pl.BlockSpec((blk, _DIM), lambda i: (i, 0)),
            out_shape=jax.ShapeDtypeStruct((_VOCAB, _DIM), jnp.float32),
        )(table)
    if _diag == "sc":
        b_pad = jnp.zeros((_DP,), jnp.float32).at[:3].set(b)
        p = jnp.zeros((_VOCAB, _DP), jnp.float32)
        idx = inputs.astype(jnp.int32).reshape(_NW, _NCHUNK, _CPW)
        mesh = plsc.VectorSubcoreMesh(core_axis_name="c",
                                      subcore_axis_name="s")
        out = pl.kernel(
            _sc_body,
            out_type=jax.ShapeDtypeStruct((_BATCH, _DP), jnp.float32),
            mesh=mesh,
            compiler_params=pltpu.CompilerParams(use_tc_tiling_on_sc=False),
            scratch_types=[
                pltpu.VMEM((_NCHUNK, _CPW), jnp.int32),
                pltpu.VMEM((_NCHUNK, _CPW, _DP), jnp.float32),
                pltpu.VMEM((_BPW, _DP), jnp.float32),
                pltpu.VMEM((_DP,), jnp.float32),
                pltpu.SemaphoreType.DMA,
            ],
        )(idx, p, b_pad)
        return out[:, :3]
    b_pad = jnp.zeros((_DP,), jnp.float32).at[:3].set(b)
    p = _project(table, w_blk).reshape(_VOCAB, _DP)
    idx = inputs.astype(jnp.int32).reshape(_NW, _NCHUNK, _CPW)
    mesh = plsc.VectorSubcoreMesh(core_axis_name="c", subcore_axis_name="s")
    out = pl.kernel(
        _sc_body,
        out_type=jax.ShapeDtypeStruct((_BATCH, _DP), jnp.float32),
        mesh=mesh,
        compiler_params=pltpu.CompilerParams(use_tc_tiling_on_sc=False),
        scratch_types=[
            pltpu.VMEM((_NCHUNK, _CPW), jnp.int32),
            pltpu.VMEM((_NCHUNK, _CPW, _DP), jnp.float32),
            pltpu.VMEM((_BPW, _DP), jnp.float32),
            pltpu.VMEM((_DP,), jnp.float32),
            pltpu.SemaphoreType.DMA,
        ],
    )(idx, p, b_pad)
    return out[:, :3]


# speedup vs baseline: 10.9843x; 1.0493x over previous
import jax
import jax.numpy as jnp
from jax import lax
from jax.experimental import pallas as pl
from jax.experimental.pallas import tpu as pltpu
from jax.experimental.pallas import tpu_sc as plsc

_VOCAB = 100000
_LEN = 50
_DIM = 64
_BATCH = 4096
_DP = 16
_NC, _NS = 2, 16
_NW = _NC * _NS
_BPW = _BATCH // _NW


_BLK = 20000
_NBLK = _VOCAB // _BLK
_G = _BLK // 8


def _proj_body(t_ref, w_ref, o_ref):
    acc = jnp.zeros((_G, 8 * _DP), jnp.float32)
    for r in range(8):
        acc = acc + jnp.dot(t_ref[0, :, r, :], w_ref[r],
                            preferred_element_type=jnp.float32)
    o_ref[...] = acc.reshape(1, _G, 8 * _DP)


def _project(table, w_blk):
    return pl.pallas_call(
        _proj_body,
        grid=(_NBLK,),
        in_specs=[
            pl.BlockSpec((1, _G, 8, _DIM), lambda i: (i, 0, 0, 0)),
            pl.BlockSpec((8, _DIM, 8 * _DP), lambda i: (0, 0, 0)),
        ],
        out_specs=pl.BlockSpec((1, _G, 8 * _DP), lambda i: (i, 0, 0)),
        out_shape=jax.ShapeDtypeStruct((_NBLK, _G, 8 * _DP), jnp.float32),
    )(table.reshape(_NBLK, _G, 8, _DIM), w_blk)


_EPC = 2
_CPW = _EPC * _LEN
_NCHUNK = _BPW // _EPC
_WIN = 8


def _sc_body(idx_hbm, p_hbm, bias_hbm, out_hbm, idx_v, rows_v, out_v, bias_v,
             sem):
    wid = lax.axis_index("s") * _NC + lax.axis_index("c")
    pltpu.sync_copy(idx_hbm.at[wid], idx_v)
    pltpu.sync_copy(bias_hbm, bias_v)
    bias = bias_v[...]

    def _fire(j):
        pltpu.async_copy(p_hbm.at[idx_v.at[j]], rows_v.at[j], sem)

    def _drain(j):
        pltpu.make_async_copy(p_hbm.at[idx_v.at[j]], rows_v.at[j],
                              sem).wait()

    def _finalize(y):
        sq = jnp.maximum(y[0] * y[0] + y[1] * y[1] + y[2] * y[2],
                         jnp.float32(1e-12))
        bits = lax.bitcast_convert_type(sq, jnp.int32)
        bits = 0x5F3759DF - lax.shift_right_logical(bits, 1)
        r = lax.bitcast_convert_type(bits, jnp.float32)
        for _ in range(3):
            r = r * (1.5 - 0.5 * sq * r * r)
        return y * r

    def _process(j):
        for e in range(_EPC):
            part = [None] * 4
            for t in range(_LEN):
                v = rows_v[j, e * _LEN + t]
                k = t % 4
                part[k] = v if part[k] is None else part[k] + v
            acc = (part[0] + part[1]) + (part[2] + part[3])
            y = acc * (1.0 / _LEN) + bias
            out_v[_EPC * j + e] = _finalize(y)

    def fire(j, c):
        _fire(j)
        return c

    def steady(j, c):
        _fire(j + _WIN)
        _drain(j)
        _process(j)
        return c

    def tail(j, c):
        _drain(j)
        _process(j)
        return c

    lax.fori_loop(0, _WIN, fire, 0)
    lax.fori_loop(0, _NCHUNK - _WIN, steady, 0)
    lax.fori_loop(_NCHUNK - _WIN, _NCHUNK, tail, 0)
    pltpu.sync_copy(out_v, out_hbm.at[pl.ds(wid * _BPW, _BPW)])


def kernel(inputs, table, W, b):
    import os as _os
    _diag = _os.environ.get("KDIAG", "")
    w_blk = jnp.zeros((8, _DIM, 8 * _DP), jnp.float32)
    for _r in range(8):
        w_blk = w_blk.at[_r, :, 16 * _r:16 * _r + 3].set(W)
    if _diag == "tcfull":
        return _project(table, w_blk)
    if _diag == "reshape":
        return table.reshape(_VOCAB // 8, 8 * _DIM)
    if _diag == "copy":
        def _cp(t_ref, o_ref):
            o_ref[...] = t_ref[...]
        blk = 10000
        return pl.pallas_call(
            _cp,
            grid=(_VOCAB // blk,),
            in_specs=[pl.BlockSpec((blk, _DIM), lambda i: (i, 0))],
            out_specs=pl.BlockSpec((blk, _DIM), lambda i: (i, 0)),
            out_shape=jax.ShapeDtypeStruct((_VOCAB, _DIM), jnp.float32),
        )(table)
    if _diag == "sc":
        b_pad = jnp.zeros((_DP,), jnp.float32).at[:3].set(b)
        p = jnp.zeros((_VOCAB, _DP), jnp.float32)
        idx = inputs.astype(jnp.int32).reshape(_NW, _NCHUNK, _CPW)
        mesh = plsc.VectorSubcoreMesh(core_axis_name="c",
                                      subcore_axis_name="s")
        out = pl.kernel(
            _sc_body,
            out_type=jax.ShapeDtypeStruct((_BATCH, _DP), jnp.float32),
            mesh=mesh,
            compiler_params=pltpu.CompilerParams(use_tc_tiling_on_sc=False),
            scratch_types=[
                pltpu.VMEM((_NCHUNK, _CPW), jnp.int32),
                pltpu.VMEM((_NCHUNK, _CPW, _DP), jnp.float32),
                pltpu.VMEM((_BPW, _DP), jnp.float32),
                pltpu.VMEM((_DP,), jnp.float32),
                pltpu.SemaphoreType.DMA,
            ],
        )(idx, p, b_pad)
        return out[:, :3]
    b_pad = jnp.zeros((_DP,), jnp.float32).at[:3].set(b)
    p = _project(table, w_blk).reshape(_VOCAB, _DP)
    idx = inputs.astype(jnp.int32).reshape(_NW, _NCHUNK, _CPW)
    mesh = plsc.VectorSubcoreMesh(core_axis_name="c", subcore_axis_name="s")
    out = pl.kernel(
        _sc_body,
        out_type=jax.ShapeDtypeStruct((_BATCH, _DP), jnp.float32),
        mesh=mesh,
        compiler_params=pltpu.CompilerParams(use_tc_tiling_on_sc=False),
        scratch_types=[
            pltpu.VMEM((_NCHUNK, _CPW), jnp.int32),
            pltpu.VMEM((_NCHUNK, _CPW, _DP), jnp.float32),
            pltpu.VMEM((_BPW, _DP), jnp.float32),
            pltpu.VMEM((_DP,), jnp.float32),
            pltpu.SemaphoreType.DMA,
        ],
    )(idx, p, b_pad)
    return out[:, :3]


# speedup vs baseline: 11.2193x; 1.0214x over previous
import jax
import jax.numpy as jnp
from jax import lax
from jax.experimental import pallas as pl
from jax.experimental.pallas import tpu as pltpu
from jax.experimental.pallas import tpu_sc as plsc

_VOCAB = 100000
_LEN = 50
_DIM = 64
_BATCH = 4096
_DP = 16
_NC, _NS = 2, 16
_NW = _NC * _NS
_BPW = _BATCH // _NW


_BLK = 20000
_NBLK = _VOCAB // _BLK
_G = _BLK // 8


def _proj_body(t_ref, w_ref, o_ref):
    acc = jnp.zeros((_G, 8 * _DP), jnp.float32)
    for r in range(8):
        acc = acc + jnp.dot(t_ref[0, :, r, :], w_ref[r],
                            preferred_element_type=jnp.float32)
    o_ref[...] = acc.reshape(1, _G, 8 * _DP)


def _project(table, w_blk):
    return pl.pallas_call(
        _proj_body,
        grid=(_NBLK,),
        in_specs=[
            pl.BlockSpec((1, _G, 8, _DIM), lambda i: (i, 0, 0, 0)),
            pl.BlockSpec((8, _DIM, 8 * _DP), lambda i: (0, 0, 0)),
        ],
        out_specs=pl.BlockSpec((1, _G, 8 * _DP), lambda i: (i, 0, 0)),
        out_shape=jax.ShapeDtypeStruct((_NBLK, _G, 8 * _DP), jnp.float32),
    )(table.reshape(_NBLK, _G, 8, _DIM), w_blk)


_EPC = 2
_CPW = _EPC * _LEN
_NCHUNK = _BPW // _EPC
_WIN = 16


def _sc_body(idx_hbm, p_hbm, bias_hbm, out_hbm, idx_v, rows_v, out_v, bias_v,
             sem):
    wid = lax.axis_index("s") * _NC + lax.axis_index("c")
    pltpu.sync_copy(idx_hbm.at[wid], idx_v)
    pltpu.sync_copy(bias_hbm, bias_v)
    bias = bias_v[...]

    def _fire(j):
        pltpu.async_copy(p_hbm.at[idx_v.at[j]], rows_v.at[j], sem)

    def _drain(j):
        pltpu.make_async_copy(p_hbm.at[idx_v.at[j]], rows_v.at[j],
                              sem).wait()

    def _finalize(y):
        sq = jnp.maximum(y[0] * y[0] + y[1] * y[1] + y[2] * y[2],
                         jnp.float32(1e-12))
        bits = lax.bitcast_convert_type(sq, jnp.int32)
        bits = 0x5F3759DF - lax.shift_right_logical(bits, 1)
        r = lax.bitcast_convert_type(bits, jnp.float32)
        for _ in range(3):
            r = r * (1.5 - 0.5 * sq * r * r)
        return y * r

    def _process(j):
        for e in range(_EPC):
            part = [None] * 4
            for t in range(_LEN):
                v = rows_v[j, e * _LEN + t]
                k = t % 4
                part[k] = v if part[k] is None else part[k] + v
            acc = (part[0] + part[1]) + (part[2] + part[3])
            y = acc * (1.0 / _LEN) + bias
            out_v[_EPC * j + e] = _finalize(y)

    def fire(j, c):
        _fire(j)
        return c

    def steady(j, c):
        _fire(j + _WIN)
        _drain(j)
        _process(j)
        return c

    def tail(j, c):
        _drain(j)
        _process(j)
        return c

    lax.fori_loop(0, _WIN, fire, 0)
    lax.fori_loop(0, _NCHUNK - _WIN, steady, 0)
    lax.fori_loop(_NCHUNK - _WIN, _NCHUNK, tail, 0)
    pltpu.sync_copy(out_v, out_hbm.at[pl.ds(wid * _BPW, _BPW)])


def kernel(inputs, table, W, b):
    import os as _os
    _diag = _os.environ.get("KDIAG", "")
    w_blk = jnp.zeros((8, _DIM, 8 * _DP), jnp.float32)
    for _r in range(8):
        w_blk = w_blk.at[_r, :, 16 * _r:16 * _r + 3].set(W)
    if _diag == "tcfull":
        return _project(table, w_blk)
    if _diag == "reshape":
        return table.reshape(_VOCAB // 8, 8 * _DIM)
    if _diag == "copy":
        def _cp(t_ref, o_ref):
            o_ref[...] = t_ref[...]
        blk = 10000
        return pl.pallas_call(
            _cp,
            grid=(_VOCAB // blk,),
            in_specs=[pl.BlockSpec((blk, _DIM), lambda i: (i, 0))],
            out_specs=pl.BlockSpec((blk, _DIM), lambda i: (i, 0)),
            out_shape=jax.ShapeDtypeStruct((_VOCAB, _DIM), jnp.float32),
        )(table)
    if _diag == "sc":
        b_pad = jnp.zeros((_DP,), jnp.float32).at[:3].set(b)
        p = jnp.zeros((_VOCAB, _DP), jnp.float32)
        idx = inputs.astype(jnp.int32).reshape(_NW, _NCHUNK, _CPW)
        mesh = plsc.VectorSubcoreMesh(core_axis_name="c",
                                      subcore_axis_name="s")
        out = pl.kernel(
            _sc_body,
            out_type=jax.ShapeDtypeStruct((_BATCH, _DP), jnp.float32),
            mesh=mesh,
            compiler_params=pltpu.CompilerParams(use_tc_tiling_on_sc=False),
            scratch_types=[
                pltpu.VMEM((_NCHUNK, _CPW), jnp.int32),
                pltpu.VMEM((_NCHUNK, _CPW, _DP), jnp.float32),
                pltpu.VMEM((_BPW, _DP), jnp.float32),
                pltpu.VMEM((_DP,), jnp.float32),
                pltpu.SemaphoreType.DMA,
            ],
        )(idx, p, b_pad)
        return out[:, :3]
    b_pad = jnp.zeros((_DP,), jnp.float32).at[:3].set(b)
    p = _project(table, w_blk).reshape(_VOCAB, _DP)
    idx = inputs.astype(jnp.int32).reshape(_NW, _NCHUNK, _CPW)
    mesh = plsc.VectorSubcoreMesh(core_axis_name="c", subcore_axis_name="s")
    out = pl.kernel(
        _sc_body,
        out_type=jax.ShapeDtypeStruct((_BATCH, _DP), jnp.float32),
        mesh=mesh,
        compiler_params=pltpu.CompilerParams(use_tc_tiling_on_sc=False),
        scratch_types=[
            pltpu.VMEM((_NCHUNK, _CPW), jnp.int32),
            pltpu.VMEM((_NCHUNK, _CPW, _DP), jnp.float32),
            pltpu.VMEM((_BPW, _DP), jnp.float32),
            pltpu.VMEM((_DP,), jnp.float32),
            pltpu.SemaphoreType.DMA,
        ],
    )(idx, p, b_pad)
    return out[:, :3]
